# baseline (speedup 1.0000x reference)
import functools

import jax
import jax.numpy as jnp
from jax import lax
from jax.experimental import pallas as pl
from jax.experimental.pallas import tpu as pltpu
from jax.experimental.pallas import tpu_sc as plsc

B, C, H, W = 8, 96, 112, 112
K = 2
NH, NW_ = H * K, W * K
PLANES = B * C
LANES = 16

RIN = 56
ROUT = RIN * K
SPLITS = H // RIN
UNITS = PLANES * SPLITS
N_WORKERS = 32
PER_WORKER = UNITS // N_WORKERS

_mesh = plsc.VectorSubcoreMesh(core_axis_name="c", subcore_axis_name="s")


def _zero_fill_even_rows(out_v, zero):
    @plsc.parallel_loop(0, ROUT // 2)
    def _(r2):
        for t in range(NW_ // LANES):
            out_v[2 * r2, pl.ds(t * LANES, LANES)] = zero


def _interleave_block(in_v, out_v, lo_idx, hi_idx, odd_mask, zero):
    @plsc.parallel_loop(0, RIN, unroll=2)
    def _(i):
        r = 2 * i + 1
        for q in range(W // LANES):
            x = in_v[i, pl.ds(q * LANES, LANES)]
            lo = jnp.where(
                odd_mask,
                jnp.take_along_axis(x, lo_idx, axis=0,
                                    mode="promise_in_bounds"),
                zero)
            hi = jnp.where(
                odd_mask,
                jnp.take_along_axis(x, hi_idx, axis=0,
                                    mode="promise_in_bounds"),
                zero)
            out_v[r, pl.ds(2 * q * LANES, LANES)] = lo
            out_v[r, pl.ds((2 * q + 1) * LANES, LANES)] = hi


@functools.partial(
    pl.kernel,
    out_type=jax.ShapeDtypeStruct((B, C, NH, NW_), jnp.float32),
    mesh=_mesh,
    scratch_types=[
        pltpu.VMEM((RIN, W), jnp.float32),
        pltpu.VMEM((RIN, W), jnp.float32),
        pltpu.VMEM((ROUT, NW_), jnp.float32),
        pltpu.VMEM((ROUT, NW_), jnp.float32),
        pltpu.SemaphoreType.DMA,
        pltpu.SemaphoreType.DMA,
        pltpu.SemaphoreType.DMA,
        pltpu.SemaphoreType.DMA,
    ],
    compiler_params=pltpu.CompilerParams(
        needs_layout_passes=False, use_tc_tiling_on_sc=True),
)
def _unpool_sc(in_hbm, out_hbm, in_v0, in_v1, out_v0, out_v1,
               sem_i0, sem_i1, sem_o0, sem_o1):
    wid = lax.axis_index("s") * 2 + lax.axis_index("c")
    base_unit = wid * PER_WORKER

    in_v = [in_v0, in_v1]
    out_v = [out_v0, out_v1]
    sem_i = [sem_i0, sem_i1]
    sem_o = [sem_o0, sem_o1]

    iota = lax.iota(jnp.int32, LANES)
    lo_idx = iota // 2
    hi_idx = lo_idx + 8
    odd_mask = (iota & 1) == 1
    zero = jnp.zeros((LANES,), jnp.float32)

    def in_slice(u):
        unit = base_unit + u
        plane = unit // SPLITS
        half = unit % SPLITS
        return in_hbm.at[plane // C, plane % C, pl.ds(half * RIN, RIN)]

    def out_slice(u):
        unit = base_unit + u
        plane = unit // SPLITS
        half = unit % SPLITS
        return out_hbm.at[plane // C, plane % C, pl.ds(half * ROUT, ROUT)]

    in_descs = [None, None]
    out_descs = [None, None]
    in_descs[0] = pltpu.async_copy(in_slice(0), in_v[0], sem_i[0])
    in_descs[1] = pltpu.async_copy(in_slice(1), in_v[1], sem_i[1])
    _zero_fill_even_rows(out_v[0], zero)
    in_descs[0].wait()
    _interleave_block(in_v[0], out_v[0], lo_idx, hi_idx, odd_mask, zero)
    out_descs[0] = pltpu.async_copy(out_v[0], out_slice(0), sem_o[0])
    _zero_fill_even_rows(out_v[1], zero)
    for p in range(1, PER_WORKER):
        u = p % 2
        if p + 1 < PER_WORKER:
            nu = (p + 1) % 2
            in_descs[nu] = pltpu.async_copy(in_slice(p + 1), in_v[nu], sem_i[nu])
        in_descs[u].wait()
        if p >= 2:
            out_descs[u].wait()
        _interleave_block(in_v[u], out_v[u], lo_idx, hi_idx, odd_mask, zero)
        out_descs[u] = pltpu.async_copy(out_v[u], out_slice(p), sem_o[u])
    out_descs[(PER_WORKER - 2) % 2].wait()
    out_descs[(PER_WORKER - 1) % 2].wait()


def kernel(input):
    return _unpool_sc(input)


# speedup vs baseline: 1.0920x; 1.0920x over previous
import functools

import jax
import jax.numpy as jnp
from jax import lax
from jax.experimental import pallas as pl
from jax.experimental.pallas import tpu as pltpu
from jax.experimental.pallas import tpu_sc as plsc

B, C, H, W = 8, 96, 112, 112
K = 2
NH, NW_ = H * K, W * K
PLANES = B * C
LANES = 16

RIN = 56
ROUT = RIN * K
SPLITS = H // RIN
UNITS = PLANES * SPLITS
N_WORKERS = 32
PER_WORKER = UNITS // N_WORKERS
RIN_H = RIN // 2
ROUT_H = ROUT // 2

_mesh = plsc.VectorSubcoreMesh(core_axis_name="c", subcore_axis_name="s")


def _zero_fill(out_v):
    zero = jnp.zeros((LANES,), jnp.float32)

    @plsc.parallel_loop(0, ROUT)
    def _(r):
        for t in range(NW_ // LANES):
            out_v[r, pl.ds(t * LANES, LANES)] = zero


def _scatter_rows(in_v, out_v, i_lo, i_hi):
    two_iota = lax.iota(jnp.int32, LANES) * 2

    @plsc.parallel_loop(i_lo, i_hi, unroll=2)
    def _(i):
        row_idx = jnp.full((LANES,), 2 * i + 1, jnp.int32)
        for q in range(W // LANES):
            x = in_v[i, pl.ds(q * LANES, LANES)]
            col_idx = two_iota + (2 * q * LANES + 1)
            plsc.store_scatter(out_v, [row_idx, col_idx], x)


@functools.partial(
    pl.kernel,
    out_type=jax.ShapeDtypeStruct((B, C, NH, NW_), jnp.float32),
    mesh=_mesh,
    scratch_types=[
        pltpu.VMEM((RIN, W), jnp.float32),
        pltpu.VMEM((RIN, W), jnp.float32),
        pltpu.VMEM((ROUT, NW_), jnp.float32),
        pltpu.VMEM((ROUT, NW_), jnp.float32),
        pltpu.SemaphoreType.DMA,
        pltpu.SemaphoreType.DMA,
        pltpu.SemaphoreType.DMA,
        pltpu.SemaphoreType.DMA,
        pltpu.SemaphoreType.DMA,
        pltpu.SemaphoreType.DMA,
    ],
    compiler_params=pltpu.CompilerParams(
        needs_layout_passes=False, use_tc_tiling_on_sc=True),
)
def _unpool_sc(in_hbm, out_hbm, in_v0, in_v1, out_v0, out_v1,
               sem_i0, sem_i1, sem_o00, sem_o01, sem_o10, sem_o11):
    wid = lax.axis_index("s") * 2 + lax.axis_index("c")
    base_unit = wid * PER_WORKER

    in_v = [in_v0, in_v1]
    out_v = [out_v0, out_v1]
    sem_i = [sem_i0, sem_i1]
    sem_o = [[sem_o00, sem_o01], [sem_o10, sem_o11]]

    def in_slice(p):
        unit = base_unit + p
        plane = unit // SPLITS
        half = unit % SPLITS
        return in_hbm.at[plane // C, plane % C, pl.ds(half * RIN, RIN)]

    def out_slice(p, hs):
        unit = base_unit + p
        plane = unit // SPLITS
        half = unit % SPLITS
        return out_hbm.at[plane // C, plane % C,
                          pl.ds(half * ROUT + hs * ROUT_H, ROUT_H)]

    def out_half(u, hs):
        return out_v[u].at[pl.ds(hs * ROUT_H, ROUT_H)]

    def process_unit(p, u, first):
        nxt = jnp.minimum(p + 1, PER_WORKER - 1)
        pltpu.async_copy(in_slice(nxt), in_v[1 - u], sem_i[1 - u])
        pltpu.make_async_copy(in_slice(p), in_v[u], sem_i[u]).wait()
        for hs in range(2):
            if not first:
                pltpu.make_async_copy(
                    out_half(u, hs), out_slice(p, hs), sem_o[u][hs]).wait()
            _scatter_rows(in_v[u], out_v[u], hs * RIN_H, (hs + 1) * RIN_H)
            pltpu.async_copy(out_half(u, hs), out_slice(p, hs), sem_o[u][hs])

    pltpu.async_copy(in_slice(0), in_v[0], sem_i[0])
    _zero_fill(out_v[0])
    _zero_fill(out_v[1])
    process_unit(jnp.int32(0), 0, True)
    process_unit(jnp.int32(1), 1, True)

    def pair_body(g, carry):
        process_unit(2 * g, 0, False)
        process_unit(2 * g + 1, 1, False)
        return carry

    lax.fori_loop(1, PER_WORKER // 2, pair_body, 0)

    pltpu.make_async_copy(
        in_slice(PER_WORKER - 1), in_v[0], sem_i[0]).wait()
    for u in range(2):
        p_last = PER_WORKER - 2 + u
        for hs in range(2):
            pltpu.make_async_copy(
                out_half(u, hs),
                out_slice(jnp.int32(p_last), hs), sem_o[u][hs]).wait()


def kernel(input):
    return _unpool_sc(input)
